# baseline (speedup 1.0000x reference)
import functools

import jax
import jax.numpy as jnp
from jax import lax
from jax.experimental import pallas as pl
from jax.experimental.pallas import tpu as pltpu
from jax.experimental.pallas import tpu_sc as plsc

N = 10000
NP = 10240
D = 256
HP = 128
B = 64
E = 160000
EB = 128
NTILE = 32
TBLK = 40
T0 = 80
EP = NTILE * TBLK * EB
EROWS = EP // EB
ROWS_T = NP // 16
NBLK = 10

@functools.lru_cache(maxsize=1)
def _sc_kernels():
    mesh = plsc.VectorSubcoreMesh(core_axis_name="c", subcore_axis_name="s",
                                  num_cores=2, num_subcores=16)

    @functools.partial(
        pl.kernel,
        out_type=jax.ShapeDtypeStruct((2, NP), jnp.float32),
        mesh=mesh,
        scratch_types=[
            pltpu.VMEM_SHARED((NP,), jnp.float32),
            pltpu.VMEM((TBLK, EB), jnp.int32),
            pltpu.VMEM((EB,), jnp.float32),
            pltpu.VMEM((ROWS_T,), jnp.float32),
        ],
    )
    def _deg_kernel(dst_hbm, deg_out, deg_sh, didx, ones_v, zbuf):
        c = lax.axis_index("c")
        s = lax.axis_index("s")
        wid = c * 16 + s
        pltpu.sync_copy(dst_hbm.at[pl.ds(wid * TBLK, TBLK)], didx)
        for k in range(EB // 16):
            ones_v[pl.ds(k * 16, 16)] = jnp.ones((16,), jnp.float32)
        for k in range(ROWS_T // 16):
            zbuf[pl.ds(k * 16, 16)] = jnp.zeros((16,), jnp.float32)
        rows = pl.ds(s * ROWS_T, ROWS_T)
        pltpu.sync_copy(zbuf, deg_sh.at[rows])
        plsc.subcore_barrier()

        def body(j, carry):
            pltpu.sync_copy(ones_v, deg_sh.at[didx.at[j]], add=True)
            return carry

        lax.fori_loop(0, TBLK, body, 0)
        plsc.subcore_barrier()
        pltpu.sync_copy(deg_sh.at[rows], deg_out.at[c, rows])

    @functools.partial(
        pl.kernel,
        out_type=jax.ShapeDtypeStruct((NP, HP), jnp.float32),
        mesh=mesh,
        scratch_types=[
            pltpu.VMEM_SHARED((NP, HP), jnp.float32),
            pltpu.VMEM((T0 // 2, EB), jnp.int32),
            pltpu.VMEM((T0 // 2, EB), jnp.int32),
            pltpu.VMEM((EB, HP), jnp.float32),
            pltpu.VMEM((EB, HP), jnp.float32),
            pltpu.SemaphoreType.DMA,
            pltpu.SemaphoreType.DMA,
        ],
    )
    def _msg_kernel(src_hbm, dst_hbm, hws_hbm, out_hbm,
                    acc_sh, sidx, didx, buf0, buf1, sem0, sem1):
        c = lax.axis_index("c")
        s = lax.axis_index("s")
        rows = pl.ds(s * ROWS_T, ROWS_T)

        @pl.when(c == 0)
        def _():
            pltpu.sync_copy(hws_hbm.at[rows], acc_sh.at[rows])
            plsc.subcore_barrier()

            HB = T0 // 2
            for h in range(2):
                pltpu.sync_copy(src_hbm.at[pl.ds(s * T0 + h * HB, HB)],
                                sidx)
                pltpu.sync_copy(dst_hbm.at[pl.ds(s * T0 + h * HB, HB)],
                                didx)
                pltpu.async_copy(hws_hbm.at[sidx.at[0]], buf0, sem0)

                def body(i, carry):
                    j = i * 2
                    pltpu.make_async_copy(hws_hbm.at[sidx.at[j]], buf0,
                                          sem0).wait()
                    pltpu.async_copy(hws_hbm.at[sidx.at[j + 1]], buf1, sem1)
                    pltpu.sync_copy(buf0, acc_sh.at[didx.at[j]], add=True)
                    pltpu.make_async_copy(hws_hbm.at[sidx.at[j + 1]], buf1,
                                          sem1).wait()

                    @pl.when(i < HB // 2 - 1)
                    def _():
                        pltpu.async_copy(hws_hbm.at[sidx.at[j + 2]], buf0,
                                         sem0)

                    pltpu.sync_copy(buf1, acc_sh.at[didx.at[j + 1]],
                                    add=True)
                    return carry

                lax.fori_loop(0, HB // 2, body, 0)
            plsc.subcore_barrier()
            pltpu.sync_copy(acc_sh.at[rows], out_hbm.at[rows])

    return _deg_kernel, _msg_kernel


def _prep_body(x_ref, g_ref, b_ref, m_ref, v_ref, w_ref, degp_ref, out_ref):
    x = x_ref[...]
    h = (x - m_ref[...]) * lax.rsqrt(v_ref[...] + 1e-5) * g_ref[...] + b_ref[...]
    hw = jnp.dot(h, w_ref[...], preferred_element_type=jnp.float32)
    deg = degp_ref[0, :] + degp_ref[1, :] + 1.0
    out_ref[...] = hw * lax.rsqrt(deg)[:, None]


def _tc_prep(x_pad, bn1g, bn1b, bn1m, bn1v, w_pad, deg_p):
    blk = NP // NBLK
    return pl.pallas_call(
        _prep_body,
        grid=(NBLK,),
        in_specs=[
            pl.BlockSpec((blk, D), lambda i: (i, 0)),
            pl.BlockSpec((1, D), lambda i: (0, 0)),
            pl.BlockSpec((1, D), lambda i: (0, 0)),
            pl.BlockSpec((1, D), lambda i: (0, 0)),
            pl.BlockSpec((1, D), lambda i: (0, 0)),
            pl.BlockSpec((D, HP), lambda i: (0, 0)),
            pl.BlockSpec((2, blk), lambda i: (0, i)),
        ],
        out_specs=pl.BlockSpec((blk, HP), lambda i: (i, 0)),
        out_shape=jax.ShapeDtypeStruct((NP, HP), jnp.float32),
    )(x_pad, bn1g, bn1b, bn1m, bn1v, w_pad, deg_p)


def _elu(x):
    return jnp.where(x > 0, x, jnp.exp(jnp.minimum(x, 0.0)) - 1.0)


def _head_body(p_ref, degp_ref, batch_ref, bg_ref,
               g2_ref, b2_ref, m2_ref, v2_ref, w1_ref, bb1_ref,
               g3_ref, b3_ref, m3_ref, v3_ref, w2_ref, bb2_ref,
               out_ref, pool_ref, cnt_ref):
    i = pl.program_id(0)

    @pl.when(i == 0)
    def _():
        pool_ref[...] = jnp.zeros_like(pool_ref)
        cnt_ref[...] = jnp.zeros_like(cnt_ref)

    deg = degp_ref[0, :] + degp_ref[1, :] + 1.0
    dinv = lax.rsqrt(deg)
    g = dinv[:, None] * p_ref[...] + bg_ref[...]
    h = _elu(g)
    bvec = batch_ref[0, 0, :]
    oh = (lax.broadcasted_iota(jnp.int32, (B, bvec.shape[0]), 0)
          == bvec[None, :]).astype(jnp.float32)
    pool_ref[...] += jnp.dot(oh, h, preferred_element_type=jnp.float32)
    cnt_ref[...] += jnp.sum(oh, axis=1)[:, None]

    @pl.when(i == NBLK - 1)
    def _():
        hp = pool_ref[...] / jnp.maximum(cnt_ref[...], 1.0)
        hb = ((hp - m2_ref[...]) * lax.rsqrt(v2_ref[...] + 1e-5)
              * g2_ref[...] + b2_ref[...])
        h1 = jnp.dot(hb, w1_ref[...], preferred_element_type=jnp.float32)
        h1 = _elu(h1 + bb1_ref[...])
        h1 = ((h1 - m3_ref[...]) * lax.rsqrt(v3_ref[...] + 1e-5)
              * g3_ref[...] + b3_ref[...])
        out_ref[...] = (jnp.dot(h1, w2_ref[...],
                                preferred_element_type=jnp.float32)
                        + bb2_ref[...])


def _tc_head(p, deg_p, batch3, bg, g2, b2, m2, v2, w1, bb1,
             g3, b3, m3, v3, w2, bb2):
    blk = NP // NBLK
    small = pl.BlockSpec((1, HP), lambda i: (0, 0))
    return pl.pallas_call(
        _head_body,
        grid=(NBLK,),
        in_specs=[
            pl.BlockSpec((blk, HP), lambda i: (i, 0)),
            pl.BlockSpec((2, blk), lambda i: (0, i)),
            pl.BlockSpec((1, 1, blk), lambda i: (i, 0, 0)),
            small, small, small, small, small,
            pl.BlockSpec((HP, HP), lambda i: (0, 0)),
            small, small, small, small, small,
            pl.BlockSpec((HP, HP), lambda i: (0, 0)),
            small,
        ],
        out_specs=pl.BlockSpec((B, HP), lambda i: (0, 0)),
        out_shape=jax.ShapeDtypeStruct((B, HP), jnp.float32),
        scratch_shapes=[
            pltpu.VMEM((B, HP), jnp.float32),
            pltpu.VMEM((B, HP), jnp.float32),
        ],
    )(p, deg_p, batch3, bg, g2, b2, m2, v2, w1, bb1,
      g3, b3, m3, v3, w2, bb2)


def _row(v, width, fill=0.0):
    out = jnp.full((1, width), fill, v.dtype)
    return lax.dynamic_update_slice(out, v[None, :], (0, 0))


def kernel(x, edge_index, batch,
           bn1_gamma, bn1_beta, bn1_mean, bn1_var,
           W_gcn, b_gcn,
           bn2_gamma, bn2_beta, bn2_mean, bn2_var,
           lin1_W, lin1_b,
           bn3_gamma, bn3_beta, bn3_mean, bn3_var,
           lin2_W, lin2_b):
    f32 = jnp.float32
    x_pad = jnp.pad(x, ((0, NP - N), (0, 0)))
    npad = EROWS * EB - E
    src = jnp.pad(edge_index[0], (0, npad)).reshape(EROWS, EB)
    pad_dst = N + jnp.arange(npad, dtype=edge_index.dtype) % (NP - N)
    dst = jnp.concatenate([edge_index[1], pad_dst]).reshape(EROWS, EB)
    batch3 = jnp.pad(batch, (0, NP - N),
                     constant_values=B).reshape(NBLK, 1, NP // NBLK)
    w_pad = jnp.pad(W_gcn, ((0, 0), (0, HP - W_gcn.shape[1])))
    bg = _row(jnp.pad(b_gcn, (0, HP - b_gcn.shape[0])), HP)
    h1 = bn2_gamma.shape[0]
    h2 = bn3_gamma.shape[0]
    g2 = _row(jnp.pad(bn2_gamma, (0, HP - h1)), HP)
    b2 = _row(jnp.pad(bn2_beta, (0, HP - h1)), HP)
    m2 = _row(jnp.pad(bn2_mean, (0, HP - h1)), HP)
    v2 = _row(jnp.pad(bn2_var, (0, HP - h1), constant_values=1.0), HP)
    w1 = jnp.pad(lin1_W, ((0, HP - h1), (0, HP - lin1_W.shape[1])))
    bb1 = _row(jnp.pad(lin1_b, (0, HP - lin1_b.shape[0])), HP)
    g3 = _row(jnp.pad(bn3_gamma, (0, HP - h2)), HP)
    b3 = _row(jnp.pad(bn3_beta, (0, HP - h2)), HP)
    m3 = _row(jnp.pad(bn3_mean, (0, HP - h2)), HP)
    v3 = _row(jnp.pad(bn3_var, (0, HP - h2), constant_values=1.0), HP)
    w2 = jnp.pad(lin2_W, ((0, HP - h2), (0, HP - lin2_W.shape[1])))
    bb2 = _row(jnp.pad(lin2_b, (0, HP - lin2_b.shape[0])), HP)
    bn1g, bn1b = bn1_gamma[None, :], bn1_beta[None, :]
    bn1m, bn1v = bn1_mean[None, :], bn1_var[None, :]
    del f32

    deg_k, msg_k = _sc_kernels()
    deg_p = deg_k(dst)
    hw_s = _tc_prep(x_pad, bn1g, bn1b, bn1m, bn1v, w_pad, deg_p)
    p = msg_k(src, dst, hw_s)
    out = _tc_head(p, deg_p, batch3, bg, g2, b2, m2, v2, w1, bb1,
                   g3, b3, m3, v3, w2, bb2)
    return out[:, :lin2_W.shape[1]]


# speedup vs baseline: 1.3825x; 1.3825x over previous
import functools

import jax
import jax.numpy as jnp
from jax import lax
from jax.experimental import pallas as pl
from jax.experimental.pallas import tpu as pltpu
from jax.experimental.pallas import tpu_sc as plsc

N = 10000
NP = 10240
D = 256
HP = 128
B = 64
E = 160000
EB = 128
NTILE = 32
TBLK = 40
T0 = 64
T1 = 2 * TBLK - T0
STAGE = max(T0, T1)
EP = NTILE * TBLK * EB
EROWS = max(EP // EB, 16 * T0 + 15 * T1 + STAGE)
ROWS_T = NP // 16
NBLK = 10

@functools.lru_cache(maxsize=1)
def _sc_kernels():
    mesh = plsc.VectorSubcoreMesh(core_axis_name="c", subcore_axis_name="s",
                                  num_cores=2, num_subcores=16)

    @functools.partial(
        pl.kernel,
        out_type=jax.ShapeDtypeStruct((2, NP), jnp.float32),
        mesh=mesh,
        scratch_types=[
            pltpu.VMEM_SHARED((NP,), jnp.float32),
            pltpu.VMEM((TBLK, EB), jnp.int32),
            pltpu.VMEM((EB,), jnp.float32),
            pltpu.VMEM((ROWS_T,), jnp.float32),
        ],
    )
    def _deg_kernel(dst_hbm, deg_out, deg_sh, didx, ones_v, zbuf):
        c = lax.axis_index("c")
        s = lax.axis_index("s")
        wid = c * 16 + s
        pltpu.sync_copy(dst_hbm.at[pl.ds(wid * TBLK, TBLK)], didx)
        for k in range(EB // 16):
            ones_v[pl.ds(k * 16, 16)] = jnp.ones((16,), jnp.float32)
        for k in range(ROWS_T // 16):
            zbuf[pl.ds(k * 16, 16)] = jnp.zeros((16,), jnp.float32)
        rows = pl.ds(s * ROWS_T, ROWS_T)
        pltpu.sync_copy(zbuf, deg_sh.at[rows])
        plsc.subcore_barrier()

        def body(j, carry):
            pltpu.sync_copy(ones_v, deg_sh.at[didx.at[j]], add=True)
            return carry

        lax.fori_loop(0, TBLK, body, 0)
        plsc.subcore_barrier()
        pltpu.sync_copy(deg_sh.at[rows], deg_out.at[c, rows])

    @functools.partial(
        pl.kernel,
        out_type=jax.ShapeDtypeStruct((2, NP, HP), jnp.float32),
        mesh=mesh,
        scratch_types=[
            pltpu.VMEM_SHARED((NP, HP), jnp.float32),
            pltpu.VMEM((STAGE, EB), jnp.int32),
            pltpu.VMEM((STAGE, EB), jnp.int32),
            pltpu.VMEM((EB, HP), jnp.float32),
            pltpu.VMEM((EB, HP), jnp.float32),
            pltpu.SemaphoreType.DMA,
            pltpu.SemaphoreType.DMA,
        ],
    )
    def _msg_kernel(src_hbm, dst_hbm, hws_hbm, out_hbm,
                    acc_sh, sidx, didx, buf0, buf1, sem0, sem1):
        c = lax.axis_index("c")
        s = lax.axis_index("s")
        rows = pl.ds(s * ROWS_T, ROWS_T)
        base = jnp.where(c == 0, s * T0, 16 * T0 + s * T1)
        pltpu.sync_copy(src_hbm.at[pl.ds(base, STAGE)], sidx)
        pltpu.sync_copy(dst_hbm.at[pl.ds(base, STAGE)], didx)

        @pl.when(c == 0)
        def _():
            pltpu.sync_copy(hws_hbm.at[rows], acc_sh.at[rows])

        @pl.when(c == 1)
        def _():
            ZR = 32
            for r in range(ZR):
                for k in range(HP // 16):
                    buf0[r, pl.ds(k * 16, 16)] = jnp.zeros((16,),
                                                           jnp.float32)
            for h in range(ROWS_T // ZR):
                pltpu.sync_copy(buf0.at[pl.ds(0, ZR)],
                                acc_sh.at[pl.ds(s * ROWS_T + h * ZR, ZR)])

        plsc.subcore_barrier()

        def pipe(nb):
            pltpu.async_copy(hws_hbm.at[sidx.at[0]], buf0, sem0)

            def body(i, carry):
                j = i * 2
                pltpu.make_async_copy(hws_hbm.at[sidx.at[j]], buf0,
                                      sem0).wait()
                pltpu.async_copy(hws_hbm.at[sidx.at[j + 1]], buf1, sem1)
                pltpu.sync_copy(buf0, acc_sh.at[didx.at[j]], add=True)
                pltpu.make_async_copy(hws_hbm.at[sidx.at[j + 1]], buf1,
                                      sem1).wait()

                @pl.when(i < nb // 2 - 1)
                def _():
                    pltpu.async_copy(hws_hbm.at[sidx.at[j + 2]], buf0, sem0)

                pltpu.sync_copy(buf1, acc_sh.at[didx.at[j + 1]], add=True)
                return carry

            lax.fori_loop(0, nb // 2, body, 0)

        @pl.when(c == 0)
        def _():
            pipe(T0)

        @pl.when(c == 1)
        def _():
            pipe(T1)

        plsc.subcore_barrier()
        pltpu.sync_copy(acc_sh.at[rows], out_hbm.at[c, rows])

    return _deg_kernel, _msg_kernel


def _prep_body(x_ref, g_ref, b_ref, m_ref, v_ref, w_ref, degp_ref, out_ref):
    x = x_ref[...]
    h = (x - m_ref[...]) * lax.rsqrt(v_ref[...] + 1e-5) * g_ref[...] + b_ref[...]
    hw = jnp.dot(h, w_ref[...], preferred_element_type=jnp.float32)
    deg = degp_ref[0, :] + degp_ref[1, :] + 1.0
    out_ref[...] = hw * lax.rsqrt(deg)[:, None]


def _tc_prep(x_pad, bn1g, bn1b, bn1m, bn1v, w_pad, deg_p):
    blk = NP // NBLK
    return pl.pallas_call(
        _prep_body,
        grid=(NBLK,),
        in_specs=[
            pl.BlockSpec((blk, D), lambda i: (i, 0)),
            pl.BlockSpec((1, D), lambda i: (0, 0)),
            pl.BlockSpec((1, D), lambda i: (0, 0)),
            pl.BlockSpec((1, D), lambda i: (0, 0)),
            pl.BlockSpec((1, D), lambda i: (0, 0)),
            pl.BlockSpec((D, HP), lambda i: (0, 0)),
            pl.BlockSpec((2, blk), lambda i: (0, i)),
        ],
        out_specs=pl.BlockSpec((blk, HP), lambda i: (i, 0)),
        out_shape=jax.ShapeDtypeStruct((NP, HP), jnp.float32),
    )(x_pad, bn1g, bn1b, bn1m, bn1v, w_pad, deg_p)


def _elu(x):
    return jnp.where(x > 0, x, jnp.exp(jnp.minimum(x, 0.0)) - 1.0)


def _head_body(p_ref, degp_ref, batch_ref, bg_ref,
               g2_ref, b2_ref, m2_ref, v2_ref, w1_ref, bb1_ref,
               g3_ref, b3_ref, m3_ref, v3_ref, w2_ref, bb2_ref,
               out_ref, pool_ref, cnt_ref):
    i = pl.program_id(0)

    @pl.when(i == 0)
    def _():
        pool_ref[...] = jnp.zeros_like(pool_ref)
        cnt_ref[...] = jnp.zeros_like(cnt_ref)

    deg = degp_ref[0, :] + degp_ref[1, :] + 1.0
    dinv = lax.rsqrt(deg)
    g = dinv[:, None] * (p_ref[0] + p_ref[1]) + bg_ref[...]
    h = _elu(g)
    bvec = batch_ref[0, 0, :]
    oh = (lax.broadcasted_iota(jnp.int32, (B, bvec.shape[0]), 0)
          == bvec[None, :]).astype(jnp.float32)
    pool_ref[...] += jnp.dot(oh, h, preferred_element_type=jnp.float32)
    cnt_ref[...] += jnp.sum(oh, axis=1)[:, None]

    @pl.when(i == NBLK - 1)
    def _():
        hp = pool_ref[...] / jnp.maximum(cnt_ref[...], 1.0)
        hb = ((hp - m2_ref[...]) * lax.rsqrt(v2_ref[...] + 1e-5)
              * g2_ref[...] + b2_ref[...])
        h1 = jnp.dot(hb, w1_ref[...], preferred_element_type=jnp.float32)
        h1 = _elu(h1 + bb1_ref[...])
        h1 = ((h1 - m3_ref[...]) * lax.rsqrt(v3_ref[...] + 1e-5)
              * g3_ref[...] + b3_ref[...])
        out_ref[...] = (jnp.dot(h1, w2_ref[...],
                                preferred_element_type=jnp.float32)
                        + bb2_ref[...])


def _tc_head(p, deg_p, batch3, bg, g2, b2, m2, v2, w1, bb1,
             g3, b3, m3, v3, w2, bb2):
    blk = NP // NBLK
    small = pl.BlockSpec((1, HP), lambda i: (0, 0))
    return pl.pallas_call(
        _head_body,
        grid=(NBLK,),
        in_specs=[
            pl.BlockSpec((2, blk, HP), lambda i: (0, i, 0)),
            pl.BlockSpec((2, blk), lambda i: (0, i)),
            pl.BlockSpec((1, 1, blk), lambda i: (i, 0, 0)),
            small, small, small, small, small,
            pl.BlockSpec((HP, HP), lambda i: (0, 0)),
            small, small, small, small, small,
            pl.BlockSpec((HP, HP), lambda i: (0, 0)),
            small,
        ],
        out_specs=pl.BlockSpec((B, HP), lambda i: (0, 0)),
        out_shape=jax.ShapeDtypeStruct((B, HP), jnp.float32),
        scratch_shapes=[
            pltpu.VMEM((B, HP), jnp.float32),
            pltpu.VMEM((B, HP), jnp.float32),
        ],
    )(p, deg_p, batch3, bg, g2, b2, m2, v2, w1, bb1,
      g3, b3, m3, v3, w2, bb2)


def _row(v, width, fill=0.0):
    out = jnp.full((1, width), fill, v.dtype)
    return lax.dynamic_update_slice(out, v[None, :], (0, 0))


def kernel(x, edge_index, batch,
           bn1_gamma, bn1_beta, bn1_mean, bn1_var,
           W_gcn, b_gcn,
           bn2_gamma, bn2_beta, bn2_mean, bn2_var,
           lin1_W, lin1_b,
           bn3_gamma, bn3_beta, bn3_mean, bn3_var,
           lin2_W, lin2_b):
    f32 = jnp.float32
    x_pad = jnp.pad(x, ((0, NP - N), (0, 0)))
    npad = EROWS * EB - E
    src = jnp.pad(edge_index[0], (0, npad)).reshape(EROWS, EB)
    pad_dst = N + jnp.arange(npad, dtype=edge_index.dtype) % (NP - N)
    dst = jnp.concatenate([edge_index[1], pad_dst]).reshape(EROWS, EB)
    batch3 = jnp.pad(batch, (0, NP - N),
                     constant_values=B).reshape(NBLK, 1, NP // NBLK)
    w_pad = jnp.pad(W_gcn, ((0, 0), (0, HP - W_gcn.shape[1])))
    bg = _row(jnp.pad(b_gcn, (0, HP - b_gcn.shape[0])), HP)
    h1 = bn2_gamma.shape[0]
    h2 = bn3_gamma.shape[0]
    g2 = _row(jnp.pad(bn2_gamma, (0, HP - h1)), HP)
    b2 = _row(jnp.pad(bn2_beta, (0, HP - h1)), HP)
    m2 = _row(jnp.pad(bn2_mean, (0, HP - h1)), HP)
    v2 = _row(jnp.pad(bn2_var, (0, HP - h1), constant_values=1.0), HP)
    w1 = jnp.pad(lin1_W, ((0, HP - h1), (0, HP - lin1_W.shape[1])))
    bb1 = _row(jnp.pad(lin1_b, (0, HP - lin1_b.shape[0])), HP)
    g3 = _row(jnp.pad(bn3_gamma, (0, HP - h2)), HP)
    b3 = _row(jnp.pad(bn3_beta, (0, HP - h2)), HP)
    m3 = _row(jnp.pad(bn3_mean, (0, HP - h2)), HP)
    v3 = _row(jnp.pad(bn3_var, (0, HP - h2), constant_values=1.0), HP)
    w2 = jnp.pad(lin2_W, ((0, HP - h2), (0, HP - lin2_W.shape[1])))
    bb2 = _row(jnp.pad(lin2_b, (0, HP - lin2_b.shape[0])), HP)
    bn1g, bn1b = bn1_gamma[None, :], bn1_beta[None, :]
    bn1m, bn1v = bn1_mean[None, :], bn1_var[None, :]
    del f32

    deg_k, msg_k = _sc_kernels()
    deg_p = deg_k(dst)
    hw_s = _tc_prep(x_pad, bn1g, bn1b, bn1m, bn1v, w_pad, deg_p)
    p = msg_k(src, dst, hw_s)
    out = _tc_head(p, deg_p, batch3, bg, g2, b2, m2, v2, w1, bb1,
                   g3, b3, m3, v3, w2, bb2)
    return out[:, :lin2_W.shape[1]]


# speedup vs baseline: 1.3894x; 1.0050x over previous
import functools

import jax
import jax.numpy as jnp
from jax import lax
from jax.experimental import pallas as pl
from jax.experimental.pallas import tpu as pltpu
from jax.experimental.pallas import tpu_sc as plsc

N = 10000
NP = 10240
D = 256
HP = 128
B = 64
E = 160000
EB = 128
NTILE = 32
TBLK = 40
T0 = 64
T1 = 2 * TBLK - T0
STAGE = max(T0, T1)
EP = NTILE * TBLK * EB
EROWS = max(EP // EB, 16 * T0 + 15 * T1 + STAGE)
ROWS_T = NP // 16
NBLK = 10

@functools.lru_cache(maxsize=1)
def _sc_kernels():
    mesh = plsc.VectorSubcoreMesh(core_axis_name="c", subcore_axis_name="s",
                                  num_cores=2, num_subcores=16)

    @functools.partial(
        pl.kernel,
        out_type=jax.ShapeDtypeStruct((2, NP), jnp.float32),
        mesh=mesh,
        scratch_types=[
            pltpu.VMEM_SHARED((NP,), jnp.float32),
            pltpu.VMEM((TBLK, EB), jnp.int32),
            pltpu.VMEM((EB,), jnp.float32),
            pltpu.VMEM((ROWS_T,), jnp.float32),
        ],
    )
    def _deg_kernel(dst_hbm, deg_out, deg_sh, didx, ones_v, zbuf):
        c = lax.axis_index("c")
        s = lax.axis_index("s")
        wid = c * 16 + s
        pltpu.sync_copy(dst_hbm.at[pl.ds(wid * TBLK, TBLK)], didx)
        for k in range(EB // 16):
            ones_v[pl.ds(k * 16, 16)] = jnp.ones((16,), jnp.float32)
        for k in range(ROWS_T // 16):
            zbuf[pl.ds(k * 16, 16)] = jnp.zeros((16,), jnp.float32)
        rows = pl.ds(s * ROWS_T, ROWS_T)
        pltpu.sync_copy(zbuf, deg_sh.at[rows])
        plsc.subcore_barrier()

        def body(j, carry):
            pltpu.sync_copy(ones_v, deg_sh.at[didx.at[j]], add=True)
            return carry

        lax.fori_loop(0, TBLK, body, 0)
        plsc.subcore_barrier()
        pltpu.sync_copy(deg_sh.at[rows], deg_out.at[c, rows])

    @functools.partial(
        pl.kernel,
        out_type=jax.ShapeDtypeStruct((2, NP, HP), jnp.float32),
        mesh=mesh,
        scratch_types=[
            pltpu.VMEM_SHARED((NP, HP), jnp.float32),
            pltpu.VMEM((STAGE, EB), jnp.int32),
            pltpu.VMEM((STAGE, EB), jnp.int32),
            pltpu.VMEM((EB, HP), jnp.float32),
            pltpu.VMEM((EB, HP), jnp.float32),
            pltpu.SemaphoreType.DMA,
            pltpu.SemaphoreType.DMA,
        ],
    )
    def _msg_kernel(src_hbm, dst_hbm, hws_hbm, out_hbm,
                    acc_sh, sidx, didx, buf0, buf1, sem0, sem1):
        c = lax.axis_index("c")
        s = lax.axis_index("s")
        rows = pl.ds(s * ROWS_T, ROWS_T)
        base = jnp.where(c == 0, s * T0, 16 * T0 + s * T1)
        pltpu.sync_copy(src_hbm.at[pl.ds(base, STAGE)], sidx)
        pltpu.sync_copy(dst_hbm.at[pl.ds(base, STAGE)], didx)

        @pl.when(c == 0)
        def _():
            pltpu.sync_copy(hws_hbm.at[rows], acc_sh.at[rows])

        @pl.when(c == 1)
        def _():
            ZR = 32
            for r in range(ZR):
                for k in range(HP // 16):
                    buf0[r, pl.ds(k * 16, 16)] = jnp.zeros((16,),
                                                           jnp.float32)
            for h in range(ROWS_T // ZR):
                pltpu.async_copy(buf0.at[pl.ds(0, ZR)],
                                 acc_sh.at[pl.ds(s * ROWS_T + h * ZR, ZR)],
                                 sem0)
            for h in range(ROWS_T // ZR):
                pltpu.make_async_copy(
                    buf0.at[pl.ds(0, ZR)],
                    acc_sh.at[pl.ds(s * ROWS_T + h * ZR, ZR)],
                    sem0).wait()

        plsc.subcore_barrier()

        def pipe(nb):
            pltpu.async_copy(hws_hbm.at[sidx.at[0]], buf0, sem0)

            def body(i, carry):
                j = i * 2
                pltpu.make_async_copy(hws_hbm.at[sidx.at[j]], buf0,
                                      sem0).wait()
                pltpu.async_copy(hws_hbm.at[sidx.at[j + 1]], buf1, sem1)
                pltpu.sync_copy(buf0, acc_sh.at[didx.at[j]], add=True)
                pltpu.make_async_copy(hws_hbm.at[sidx.at[j + 1]], buf1,
                                      sem1).wait()

                @pl.when(i < nb // 2 - 1)
                def _():
                    pltpu.async_copy(hws_hbm.at[sidx.at[j + 2]], buf0, sem0)

                pltpu.sync_copy(buf1, acc_sh.at[didx.at[j + 1]], add=True)
                return carry

            lax.fori_loop(0, nb // 2, body, 0)

        @pl.when(c == 0)
        def _():
            pipe(T0)

        @pl.when(c == 1)
        def _():
            pipe(T1)

        plsc.subcore_barrier()
        pltpu.sync_copy(acc_sh.at[rows], out_hbm.at[c, rows])

    return _deg_kernel, _msg_kernel


def _prep_body(x_ref, g_ref, b_ref, m_ref, v_ref, w_ref, degp_ref, out_ref):
    x = x_ref[...]
    h = (x - m_ref[...]) * lax.rsqrt(v_ref[...] + 1e-5) * g_ref[...] + b_ref[...]
    hw = jnp.dot(h, w_ref[...], preferred_element_type=jnp.float32)
    deg = degp_ref[0, :] + degp_ref[1, :] + 1.0
    out_ref[...] = hw * lax.rsqrt(deg)[:, None]


def _tc_prep(x_pad, bn1g, bn1b, bn1m, bn1v, w_pad, deg_p):
    blk = NP // NBLK
    return pl.pallas_call(
        _prep_body,
        grid=(NBLK,),
        in_specs=[
            pl.BlockSpec((blk, D), lambda i: (i, 0)),
            pl.BlockSpec((1, D), lambda i: (0, 0)),
            pl.BlockSpec((1, D), lambda i: (0, 0)),
            pl.BlockSpec((1, D), lambda i: (0, 0)),
            pl.BlockSpec((1, D), lambda i: (0, 0)),
            pl.BlockSpec((D, HP), lambda i: (0, 0)),
            pl.BlockSpec((2, blk), lambda i: (0, i)),
        ],
        out_specs=pl.BlockSpec((blk, HP), lambda i: (i, 0)),
        out_shape=jax.ShapeDtypeStruct((NP, HP), jnp.float32),
    )(x_pad, bn1g, bn1b, bn1m, bn1v, w_pad, deg_p)


def _elu(x):
    return jnp.where(x > 0, x, jnp.exp(jnp.minimum(x, 0.0)) - 1.0)


def _head_body(p_ref, degp_ref, batch_ref, bg_ref,
               g2_ref, b2_ref, m2_ref, v2_ref, w1_ref, bb1_ref,
               g3_ref, b3_ref, m3_ref, v3_ref, w2_ref, bb2_ref,
               out_ref, pool_ref, cnt_ref):
    i = pl.program_id(0)

    @pl.when(i == 0)
    def _():
        pool_ref[...] = jnp.zeros_like(pool_ref)
        cnt_ref[...] = jnp.zeros_like(cnt_ref)

    deg = degp_ref[0, :] + degp_ref[1, :] + 1.0
    dinv = lax.rsqrt(deg)
    g = dinv[:, None] * (p_ref[0] + p_ref[1]) + bg_ref[...]
    h = _elu(g)
    bvec = batch_ref[0, 0, :]
    oh = (lax.broadcasted_iota(jnp.int32, (B, bvec.shape[0]), 0)
          == bvec[None, :]).astype(jnp.float32)
    pool_ref[...] += jnp.dot(oh, h, preferred_element_type=jnp.float32)
    cnt_ref[...] += jnp.sum(oh, axis=1)[:, None]

    @pl.when(i == NBLK - 1)
    def _():
        hp = pool_ref[...] / jnp.maximum(cnt_ref[...], 1.0)
        hb = ((hp - m2_ref[...]) * lax.rsqrt(v2_ref[...] + 1e-5)
              * g2_ref[...] + b2_ref[...])
        h1 = jnp.dot(hb, w1_ref[...], preferred_element_type=jnp.float32)
        h1 = _elu(h1 + bb1_ref[...])
        h1 = ((h1 - m3_ref[...]) * lax.rsqrt(v3_ref[...] + 1e-5)
              * g3_ref[...] + b3_ref[...])
        out_ref[...] = (jnp.dot(h1, w2_ref[...],
                                preferred_element_type=jnp.float32)
                        + bb2_ref[...])


def _tc_head(p, deg_p, batch3, bg, g2, b2, m2, v2, w1, bb1,
             g3, b3, m3, v3, w2, bb2):
    blk = NP // NBLK
    small = pl.BlockSpec((1, HP), lambda i: (0, 0))
    return pl.pallas_call(
        _head_body,
        grid=(NBLK,),
        in_specs=[
            pl.BlockSpec((2, blk, HP), lambda i: (0, i, 0)),
            pl.BlockSpec((2, blk), lambda i: (0, i)),
            pl.BlockSpec((1, 1, blk), lambda i: (i, 0, 0)),
            small, small, small, small, small,
            pl.BlockSpec((HP, HP), lambda i: (0, 0)),
            small, small, small, small, small,
            pl.BlockSpec((HP, HP), lambda i: (0, 0)),
            small,
        ],
        out_specs=pl.BlockSpec((B, HP), lambda i: (0, 0)),
        out_shape=jax.ShapeDtypeStruct((B, HP), jnp.float32),
        scratch_shapes=[
            pltpu.VMEM((B, HP), jnp.float32),
            pltpu.VMEM((B, HP), jnp.float32),
        ],
    )(p, deg_p, batch3, bg, g2, b2, m2, v2, w1, bb1,
      g3, b3, m3, v3, w2, bb2)


def _row(v, width, fill=0.0):
    out = jnp.full((1, width), fill, v.dtype)
    return lax.dynamic_update_slice(out, v[None, :], (0, 0))


def kernel(x, edge_index, batch,
           bn1_gamma, bn1_beta, bn1_mean, bn1_var,
           W_gcn, b_gcn,
           bn2_gamma, bn2_beta, bn2_mean, bn2_var,
           lin1_W, lin1_b,
           bn3_gamma, bn3_beta, bn3_mean, bn3_var,
           lin2_W, lin2_b):
    f32 = jnp.float32
    x_pad = jnp.pad(x, ((0, NP - N), (0, 0)))
    npad = EROWS * EB - E
    src = jnp.pad(edge_index[0], (0, npad)).reshape(EROWS, EB)
    pad_dst = N + jnp.arange(npad, dtype=edge_index.dtype) % (NP - N)
    dst = jnp.concatenate([edge_index[1], pad_dst]).reshape(EROWS, EB)
    batch3 = jnp.pad(batch, (0, NP - N),
                     constant_values=B).reshape(NBLK, 1, NP // NBLK)
    w_pad = jnp.pad(W_gcn, ((0, 0), (0, HP - W_gcn.shape[1])))
    bg = _row(jnp.pad(b_gcn, (0, HP - b_gcn.shape[0])), HP)
    h1 = bn2_gamma.shape[0]
    h2 = bn3_gamma.shape[0]
    g2 = _row(jnp.pad(bn2_gamma, (0, HP - h1)), HP)
    b2 = _row(jnp.pad(bn2_beta, (0, HP - h1)), HP)
    m2 = _row(jnp.pad(bn2_mean, (0, HP - h1)), HP)
    v2 = _row(jnp.pad(bn2_var, (0, HP - h1), constant_values=1.0), HP)
    w1 = jnp.pad(lin1_W, ((0, HP - h1), (0, HP - lin1_W.shape[1])))
    bb1 = _row(jnp.pad(lin1_b, (0, HP - lin1_b.shape[0])), HP)
    g3 = _row(jnp.pad(bn3_gamma, (0, HP - h2)), HP)
    b3 = _row(jnp.pad(bn3_beta, (0, HP - h2)), HP)
    m3 = _row(jnp.pad(bn3_mean, (0, HP - h2)), HP)
    v3 = _row(jnp.pad(bn3_var, (0, HP - h2), constant_values=1.0), HP)
    w2 = jnp.pad(lin2_W, ((0, HP - h2), (0, HP - lin2_W.shape[1])))
    bb2 = _row(jnp.pad(lin2_b, (0, HP - lin2_b.shape[0])), HP)
    bn1g, bn1b = bn1_gamma[None, :], bn1_beta[None, :]
    bn1m, bn1v = bn1_mean[None, :], bn1_var[None, :]
    del f32

    deg_k, msg_k = _sc_kernels()
    deg_p = deg_k(dst)
    hw_s = _tc_prep(x_pad, bn1g, bn1b, bn1m, bn1v, w_pad, deg_p)
    p = msg_k(src, dst, hw_s)
    out = _tc_head(p, deg_p, batch3, bg, g2, b2, m2, v2, w1, bb1,
                   g3, b3, m3, v3, w2, bb2)
    return out[:, :lin2_W.shape[1]]


# speedup vs baseline: 1.4453x; 1.0403x over previous
import functools

import jax
import jax.numpy as jnp
from jax import lax
from jax.experimental import pallas as pl
from jax.experimental.pallas import tpu as pltpu
from jax.experimental.pallas import tpu_sc as plsc

N = 10000
NP = 10240
D = 256
HP = 128
B = 64
E = 160000
EB = 128
NTILE = 32
TBLK = 40
T0 = 64
T1 = 2 * TBLK - T0
STAGE = max(T0, T1)
EP = NTILE * TBLK * EB
EROWS = max(EP // EB, 16 * T0 + 15 * T1 + STAGE)
ROWS_T = NP // 16
NBLK = 10

@functools.lru_cache(maxsize=1)
def _sc_kernels():
    mesh = plsc.VectorSubcoreMesh(core_axis_name="c", subcore_axis_name="s",
                                  num_cores=2, num_subcores=16)

    @functools.partial(
        pl.kernel,
        out_type=jax.ShapeDtypeStruct((2, NP), jnp.float32),
        mesh=mesh,
        scratch_types=[
            pltpu.VMEM_SHARED((NP,), jnp.float32),
            pltpu.VMEM((TBLK, EB), jnp.int32),
            pltpu.VMEM((EB,), jnp.float32),
            pltpu.VMEM((ROWS_T,), jnp.float32),
        ],
    )
    def _deg_kernel(dst_hbm, deg_out, deg_sh, didx, ones_v, zbuf):
        c = lax.axis_index("c")
        s = lax.axis_index("s")
        wid = c * 16 + s
        pltpu.sync_copy(dst_hbm.at[pl.ds(wid * TBLK, TBLK)], didx)
        for k in range(EB // 16):
            ones_v[pl.ds(k * 16, 16)] = jnp.ones((16,), jnp.float32)
        for k in range(ROWS_T // 16):
            zbuf[pl.ds(k * 16, 16)] = jnp.zeros((16,), jnp.float32)
        rows = pl.ds(s * ROWS_T, ROWS_T)
        pltpu.sync_copy(zbuf, deg_sh.at[rows])
        plsc.subcore_barrier()

        def body(j, carry):
            pltpu.sync_copy(ones_v, deg_sh.at[didx.at[j]], add=True)
            return carry

        lax.fori_loop(0, TBLK, body, 0)
        plsc.subcore_barrier()
        pltpu.sync_copy(deg_sh.at[rows], deg_out.at[c, rows])

    @functools.partial(
        pl.kernel,
        out_type=jax.ShapeDtypeStruct((2, NP, HP), jnp.float32),
        mesh=mesh,
        scratch_types=[
            pltpu.VMEM_SHARED((NP, HP), jnp.float32),
            pltpu.VMEM((STAGE, EB), jnp.int32),
            pltpu.VMEM((STAGE, EB), jnp.int32),
            pltpu.VMEM((EB, HP), jnp.float32),
            pltpu.VMEM((EB, HP), jnp.float32),
            pltpu.SemaphoreType.DMA,
            pltpu.SemaphoreType.DMA,
        ],
    )
    def _msg_kernel(src_hbm, dst_hbm, hws_hbm, zeros_hbm, out_hbm,
                    acc_sh, sidx, didx, buf0, buf1, sem0, sem1):
        c = lax.axis_index("c")
        s = lax.axis_index("s")
        rows = pl.ds(s * ROWS_T, ROWS_T)
        base = jnp.where(c == 0, s * T0, 16 * T0 + s * T1)
        pltpu.sync_copy(src_hbm.at[pl.ds(base, STAGE)], sidx)
        pltpu.sync_copy(dst_hbm.at[pl.ds(base, STAGE)], didx)

        @pl.when(c == 0)
        def _():
            pltpu.sync_copy(hws_hbm.at[rows], acc_sh.at[rows])

        @pl.when(c == 1)
        def _():
            pltpu.sync_copy(zeros_hbm.at[rows], acc_sh.at[rows])

        plsc.subcore_barrier()

        def pipe(nb):
            pltpu.async_copy(hws_hbm.at[sidx.at[0]], buf0, sem0)

            def body(i, carry):
                j = i * 2
                pltpu.make_async_copy(hws_hbm.at[sidx.at[j]], buf0,
                                      sem0).wait()
                pltpu.async_copy(hws_hbm.at[sidx.at[j + 1]], buf1, sem1)
                pltpu.sync_copy(buf0, acc_sh.at[didx.at[j]], add=True)
                pltpu.make_async_copy(hws_hbm.at[sidx.at[j + 1]], buf1,
                                      sem1).wait()

                @pl.when(i < nb // 2 - 1)
                def _():
                    pltpu.async_copy(hws_hbm.at[sidx.at[j + 2]], buf0, sem0)

                pltpu.sync_copy(buf1, acc_sh.at[didx.at[j + 1]], add=True)
                return carry

            lax.fori_loop(0, nb // 2, body, 0)

        @pl.when(c == 0)
        def _():
            pipe(T0)

        @pl.when(c == 1)
        def _():
            pipe(T1)

        plsc.subcore_barrier()
        pltpu.sync_copy(acc_sh.at[rows], out_hbm.at[c, rows])

    return _deg_kernel, _msg_kernel


def _prep_body(x_ref, g_ref, b_ref, m_ref, v_ref, w_ref, degp_ref, out_ref):
    x = x_ref[...]
    h = (x - m_ref[...]) * lax.rsqrt(v_ref[...] + 1e-5) * g_ref[...] + b_ref[...]
    hw = jnp.dot(h, w_ref[...], preferred_element_type=jnp.float32)
    deg = degp_ref[0, :] + degp_ref[1, :] + 1.0
    out_ref[...] = hw * lax.rsqrt(deg)[:, None]


def _tc_prep(x_pad, bn1g, bn1b, bn1m, bn1v, w_pad, deg_p):
    blk = NP // NBLK
    return pl.pallas_call(
        _prep_body,
        grid=(NBLK,),
        in_specs=[
            pl.BlockSpec((blk, D), lambda i: (i, 0)),
            pl.BlockSpec((1, D), lambda i: (0, 0)),
            pl.BlockSpec((1, D), lambda i: (0, 0)),
            pl.BlockSpec((1, D), lambda i: (0, 0)),
            pl.BlockSpec((1, D), lambda i: (0, 0)),
            pl.BlockSpec((D, HP), lambda i: (0, 0)),
            pl.BlockSpec((2, blk), lambda i: (0, i)),
        ],
        out_specs=pl.BlockSpec((blk, HP), lambda i: (i, 0)),
        out_shape=jax.ShapeDtypeStruct((NP, HP), jnp.float32),
    )(x_pad, bn1g, bn1b, bn1m, bn1v, w_pad, deg_p)


def _elu(x):
    return jnp.where(x > 0, x, jnp.exp(jnp.minimum(x, 0.0)) - 1.0)


def _head_body(p_ref, degp_ref, batch_ref, bg_ref,
               g2_ref, b2_ref, m2_ref, v2_ref, w1_ref, bb1_ref,
               g3_ref, b3_ref, m3_ref, v3_ref, w2_ref, bb2_ref,
               out_ref, pool_ref, cnt_ref):
    i = pl.program_id(0)

    @pl.when(i == 0)
    def _():
        pool_ref[...] = jnp.zeros_like(pool_ref)
        cnt_ref[...] = jnp.zeros_like(cnt_ref)

    deg = degp_ref[0, :] + degp_ref[1, :] + 1.0
    dinv = lax.rsqrt(deg)
    g = dinv[:, None] * (p_ref[0] + p_ref[1]) + bg_ref[...]
    h = _elu(g)
    bvec = batch_ref[0, 0, :]
    oh = (lax.broadcasted_iota(jnp.int32, (B, bvec.shape[0]), 0)
          == bvec[None, :]).astype(jnp.float32)
    pool_ref[...] += jnp.dot(oh, h, preferred_element_type=jnp.float32)
    cnt_ref[...] += jnp.sum(oh, axis=1)[:, None]

    @pl.when(i == NBLK - 1)
    def _():
        hp = pool_ref[...] / jnp.maximum(cnt_ref[...], 1.0)
        hb = ((hp - m2_ref[...]) * lax.rsqrt(v2_ref[...] + 1e-5)
              * g2_ref[...] + b2_ref[...])
        h1 = jnp.dot(hb, w1_ref[...], preferred_element_type=jnp.float32)
        h1 = _elu(h1 + bb1_ref[...])
        h1 = ((h1 - m3_ref[...]) * lax.rsqrt(v3_ref[...] + 1e-5)
              * g3_ref[...] + b3_ref[...])
        out_ref[...] = (jnp.dot(h1, w2_ref[...],
                                preferred_element_type=jnp.float32)
                        + bb2_ref[...])


def _tc_head(p, deg_p, batch3, bg, g2, b2, m2, v2, w1, bb1,
             g3, b3, m3, v3, w2, bb2):
    blk = NP // NBLK
    small = pl.BlockSpec((1, HP), lambda i: (0, 0))
    return pl.pallas_call(
        _head_body,
        grid=(NBLK,),
        in_specs=[
            pl.BlockSpec((2, blk, HP), lambda i: (0, i, 0)),
            pl.BlockSpec((2, blk), lambda i: (0, i)),
            pl.BlockSpec((1, 1, blk), lambda i: (i, 0, 0)),
            small, small, small, small, small,
            pl.BlockSpec((HP, HP), lambda i: (0, 0)),
            small, small, small, small, small,
            pl.BlockSpec((HP, HP), lambda i: (0, 0)),
            small,
        ],
        out_specs=pl.BlockSpec((B, HP), lambda i: (0, 0)),
        out_shape=jax.ShapeDtypeStruct((B, HP), jnp.float32),
        scratch_shapes=[
            pltpu.VMEM((B, HP), jnp.float32),
            pltpu.VMEM((B, HP), jnp.float32),
        ],
    )(p, deg_p, batch3, bg, g2, b2, m2, v2, w1, bb1,
      g3, b3, m3, v3, w2, bb2)


def _row(v, width, fill=0.0):
    out = jnp.full((1, width), fill, v.dtype)
    return lax.dynamic_update_slice(out, v[None, :], (0, 0))


def kernel(x, edge_index, batch,
           bn1_gamma, bn1_beta, bn1_mean, bn1_var,
           W_gcn, b_gcn,
           bn2_gamma, bn2_beta, bn2_mean, bn2_var,
           lin1_W, lin1_b,
           bn3_gamma, bn3_beta, bn3_mean, bn3_var,
           lin2_W, lin2_b):
    f32 = jnp.float32
    x_pad = jnp.pad(x, ((0, NP - N), (0, 0)))
    npad = EROWS * EB - E
    src = jnp.pad(edge_index[0], (0, npad)).reshape(EROWS, EB)
    pad_dst = N + jnp.arange(npad, dtype=edge_index.dtype) % (NP - N)
    dst = jnp.concatenate([edge_index[1], pad_dst]).reshape(EROWS, EB)
    batch3 = jnp.pad(batch, (0, NP - N),
                     constant_values=B).reshape(NBLK, 1, NP // NBLK)
    w_pad = jnp.pad(W_gcn, ((0, 0), (0, HP - W_gcn.shape[1])))
    bg = _row(jnp.pad(b_gcn, (0, HP - b_gcn.shape[0])), HP)
    h1 = bn2_gamma.shape[0]
    h2 = bn3_gamma.shape[0]
    g2 = _row(jnp.pad(bn2_gamma, (0, HP - h1)), HP)
    b2 = _row(jnp.pad(bn2_beta, (0, HP - h1)), HP)
    m2 = _row(jnp.pad(bn2_mean, (0, HP - h1)), HP)
    v2 = _row(jnp.pad(bn2_var, (0, HP - h1), constant_values=1.0), HP)
    w1 = jnp.pad(lin1_W, ((0, HP - h1), (0, HP - lin1_W.shape[1])))
    bb1 = _row(jnp.pad(lin1_b, (0, HP - lin1_b.shape[0])), HP)
    g3 = _row(jnp.pad(bn3_gamma, (0, HP - h2)), HP)
    b3 = _row(jnp.pad(bn3_beta, (0, HP - h2)), HP)
    m3 = _row(jnp.pad(bn3_mean, (0, HP - h2)), HP)
    v3 = _row(jnp.pad(bn3_var, (0, HP - h2), constant_values=1.0), HP)
    w2 = jnp.pad(lin2_W, ((0, HP - h2), (0, HP - lin2_W.shape[1])))
    bb2 = _row(jnp.pad(lin2_b, (0, HP - lin2_b.shape[0])), HP)
    bn1g, bn1b = bn1_gamma[None, :], bn1_beta[None, :]
    bn1m, bn1v = bn1_mean[None, :], bn1_var[None, :]
    zeros_nh = jnp.zeros((NP, HP), f32)

    deg_k, msg_k = _sc_kernels()
    deg_p = deg_k(dst)
    hw_s = _tc_prep(x_pad, bn1g, bn1b, bn1m, bn1v, w_pad, deg_p)
    p = msg_k(src, dst, hw_s, zeros_nh)
    out = _tc_head(p, deg_p, batch3, bg, g2, b2, m2, v2, w1, bb1,
                   g3, b3, m3, v3, w2, bb2)
    return out[:, :lin2_W.shape[1]]


# speedup vs baseline: 1.4684x; 1.0160x over previous
import functools

import jax
import jax.numpy as jnp
from jax import lax
from jax.experimental import pallas as pl
from jax.experimental.pallas import tpu as pltpu
from jax.experimental.pallas import tpu_sc as plsc

N = 10000
NP = 10240
D = 256
HP = 128
B = 64
E = 160000
EB = 128
NTILE = 32
TBLK = 40
T0 = 72
T1 = 2 * TBLK - T0
STAGE = max(T0, T1)
DHALVES = ((0, 40), (40, 32)) if T0 == 72 else ((0, T0),)
DSTG = max(h[1] for h in DHALVES)
EP = NTILE * TBLK * EB
EROWS = max(EP // EB, 16 * T0 + 15 * T1 + STAGE)
ROWS_T = NP // 16
NBLK = 10

@functools.lru_cache(maxsize=1)
def _sc_kernels():
    mesh = plsc.VectorSubcoreMesh(core_axis_name="c", subcore_axis_name="s",
                                  num_cores=2, num_subcores=16)

    @functools.partial(
        pl.kernel,
        out_type=jax.ShapeDtypeStruct((2, NP), jnp.float32),
        mesh=mesh,
        scratch_types=[
            pltpu.VMEM_SHARED((NP,), jnp.float32),
            pltpu.VMEM((TBLK, EB), jnp.int32),
            pltpu.VMEM((EB,), jnp.float32),
            pltpu.VMEM((ROWS_T,), jnp.float32),
        ],
    )
    def _deg_kernel(dst_hbm, deg_out, deg_sh, didx, ones_v, zbuf):
        c = lax.axis_index("c")
        s = lax.axis_index("s")
        wid = c * 16 + s
        pltpu.sync_copy(dst_hbm.at[pl.ds(wid * TBLK, TBLK)], didx)
        for k in range(EB // 16):
            ones_v[pl.ds(k * 16, 16)] = jnp.ones((16,), jnp.float32)
        for k in range(ROWS_T // 16):
            zbuf[pl.ds(k * 16, 16)] = jnp.zeros((16,), jnp.float32)
        rows = pl.ds(s * ROWS_T, ROWS_T)
        pltpu.sync_copy(zbuf, deg_sh.at[rows])
        plsc.subcore_barrier()

        def body(j, carry):
            pltpu.sync_copy(ones_v, deg_sh.at[didx.at[j]], add=True)
            return carry

        lax.fori_loop(0, TBLK, body, 0)
        plsc.subcore_barrier()
        pltpu.sync_copy(deg_sh.at[rows], deg_out.at[c, rows])

    @functools.partial(
        pl.kernel,
        out_type=jax.ShapeDtypeStruct((2, NP, HP), jnp.float32),
        mesh=mesh,
        scratch_types=[
            pltpu.VMEM_SHARED((NP, HP), jnp.float32),
            pltpu.VMEM((STAGE, EB), jnp.int32),
            pltpu.VMEM((DSTG, EB), jnp.int32),
            pltpu.VMEM((EB, HP), jnp.float32),
            pltpu.VMEM((EB, HP), jnp.float32),
            pltpu.SemaphoreType.DMA,
            pltpu.SemaphoreType.DMA,
        ],
    )
    def _msg_kernel(src_hbm, dst_hbm, hws_hbm, zeros_hbm, out_hbm,
                    acc_sh, sidx, didx, buf0, buf1, sem0, sem1):
        c = lax.axis_index("c")
        s = lax.axis_index("s")
        rows = pl.ds(s * ROWS_T, ROWS_T)
        base = jnp.where(c == 0, s * T0, 16 * T0 + s * T1)
        pltpu.sync_copy(src_hbm.at[pl.ds(base, STAGE)], sidx)

        @pl.when(c == 0)
        def _():
            pltpu.sync_copy(hws_hbm.at[rows], acc_sh.at[rows])

        @pl.when(c == 1)
        def _():
            pltpu.sync_copy(zeros_hbm.at[rows], acc_sh.at[rows])

        plsc.subcore_barrier()

        def pipe(halves):
            for h0, hn in halves:
                pltpu.sync_copy(dst_hbm.at[pl.ds(base + h0, hn)],
                                didx.at[pl.ds(0, hn)])
                pltpu.async_copy(hws_hbm.at[sidx.at[h0]], buf0, sem0)

                def body(i, carry):
                    j = h0 + i * 2
                    jl = i * 2
                    pltpu.make_async_copy(hws_hbm.at[sidx.at[j]], buf0,
                                          sem0).wait()
                    pltpu.async_copy(hws_hbm.at[sidx.at[j + 1]], buf1, sem1)
                    pltpu.sync_copy(buf0, acc_sh.at[didx.at[jl]], add=True)
                    pltpu.make_async_copy(hws_hbm.at[sidx.at[j + 1]], buf1,
                                          sem1).wait()

                    @pl.when(i < hn // 2 - 1)
                    def _():
                        pltpu.async_copy(hws_hbm.at[sidx.at[j + 2]], buf0,
                                         sem0)

                    pltpu.sync_copy(buf1, acc_sh.at[didx.at[jl + 1]],
                                    add=True)
                    return carry

                lax.fori_loop(0, hn // 2, body, 0)

        @pl.when(c == 0)
        def _():
            pipe(DHALVES)

        @pl.when(c == 1)
        def _():
            pipe(((0, T1),))

        plsc.subcore_barrier()
        pltpu.sync_copy(acc_sh.at[rows], out_hbm.at[c, rows])

    return _deg_kernel, _msg_kernel


def _prep_body(x_ref, g_ref, b_ref, m_ref, v_ref, w_ref, degp_ref, out_ref):
    x = x_ref[...]
    h = (x - m_ref[...]) * lax.rsqrt(v_ref[...] + 1e-5) * g_ref[...] + b_ref[...]
    hw = jnp.dot(h, w_ref[...], preferred_element_type=jnp.float32)
    deg = degp_ref[0, :] + degp_ref[1, :] + 1.0
    out_ref[...] = hw * lax.rsqrt(deg)[:, None]


def _tc_prep(x_pad, bn1g, bn1b, bn1m, bn1v, w_pad, deg_p):
    blk = NP // NBLK
    return pl.pallas_call(
        _prep_body,
        grid=(NBLK,),
        in_specs=[
            pl.BlockSpec((blk, D), lambda i: (i, 0)),
            pl.BlockSpec((1, D), lambda i: (0, 0)),
            pl.BlockSpec((1, D), lambda i: (0, 0)),
            pl.BlockSpec((1, D), lambda i: (0, 0)),
            pl.BlockSpec((1, D), lambda i: (0, 0)),
            pl.BlockSpec((D, HP), lambda i: (0, 0)),
            pl.BlockSpec((2, blk), lambda i: (0, i)),
        ],
        out_specs=pl.BlockSpec((blk, HP), lambda i: (i, 0)),
        out_shape=jax.ShapeDtypeStruct((NP, HP), jnp.float32),
    )(x_pad, bn1g, bn1b, bn1m, bn1v, w_pad, deg_p)


def _elu(x):
    return jnp.where(x > 0, x, jnp.exp(jnp.minimum(x, 0.0)) - 1.0)


def _head_body(p_ref, degp_ref, batch_ref, bg_ref,
               g2_ref, b2_ref, m2_ref, v2_ref, w1_ref, bb1_ref,
               g3_ref, b3_ref, m3_ref, v3_ref, w2_ref, bb2_ref,
               out_ref, pool_ref, cnt_ref):
    i = pl.program_id(0)

    @pl.when(i == 0)
    def _():
        pool_ref[...] = jnp.zeros_like(pool_ref)
        cnt_ref[...] = jnp.zeros_like(cnt_ref)

    deg = degp_ref[0, :] + degp_ref[1, :] + 1.0
    dinv = lax.rsqrt(deg)
    g = dinv[:, None] * (p_ref[0] + p_ref[1]) + bg_ref[...]
    h = _elu(g)
    bvec = batch_ref[0, 0, :]
    oh = (lax.broadcasted_iota(jnp.int32, (B, bvec.shape[0]), 0)
          == bvec[None, :]).astype(jnp.float32)
    pool_ref[...] += jnp.dot(oh, h, preferred_element_type=jnp.float32)
    cnt_ref[...] += jnp.sum(oh, axis=1)[:, None]

    @pl.when(i == NBLK - 1)
    def _():
        hp = pool_ref[...] / jnp.maximum(cnt_ref[...], 1.0)
        hb = ((hp - m2_ref[...]) * lax.rsqrt(v2_ref[...] + 1e-5)
              * g2_ref[...] + b2_ref[...])
        h1 = jnp.dot(hb, w1_ref[...], preferred_element_type=jnp.float32)
        h1 = _elu(h1 + bb1_ref[...])
        h1 = ((h1 - m3_ref[...]) * lax.rsqrt(v3_ref[...] + 1e-5)
              * g3_ref[...] + b3_ref[...])
        out_ref[...] = (jnp.dot(h1, w2_ref[...],
                                preferred_element_type=jnp.float32)
                        + bb2_ref[...])


def _tc_head(p, deg_p, batch3, bg, g2, b2, m2, v2, w1, bb1,
             g3, b3, m3, v3, w2, bb2):
    blk = NP // NBLK
    small = pl.BlockSpec((1, HP), lambda i: (0, 0))
    return pl.pallas_call(
        _head_body,
        grid=(NBLK,),
        in_specs=[
            pl.BlockSpec((2, blk, HP), lambda i: (0, i, 0)),
            pl.BlockSpec((2, blk), lambda i: (0, i)),
            pl.BlockSpec((1, 1, blk), lambda i: (i, 0, 0)),
            small, small, small, small, small,
            pl.BlockSpec((HP, HP), lambda i: (0, 0)),
            small, small, small, small, small,
            pl.BlockSpec((HP, HP), lambda i: (0, 0)),
            small,
        ],
        out_specs=pl.BlockSpec((B, HP), lambda i: (0, 0)),
        out_shape=jax.ShapeDtypeStruct((B, HP), jnp.float32),
        scratch_shapes=[
            pltpu.VMEM((B, HP), jnp.float32),
            pltpu.VMEM((B, HP), jnp.float32),
        ],
    )(p, deg_p, batch3, bg, g2, b2, m2, v2, w1, bb1,
      g3, b3, m3, v3, w2, bb2)


def _row(v, width, fill=0.0):
    out = jnp.full((1, width), fill, v.dtype)
    return lax.dynamic_update_slice(out, v[None, :], (0, 0))


def kernel(x, edge_index, batch,
           bn1_gamma, bn1_beta, bn1_mean, bn1_var,
           W_gcn, b_gcn,
           bn2_gamma, bn2_beta, bn2_mean, bn2_var,
           lin1_W, lin1_b,
           bn3_gamma, bn3_beta, bn3_mean, bn3_var,
           lin2_W, lin2_b):
    f32 = jnp.float32
    x_pad = jnp.pad(x, ((0, NP - N), (0, 0)))
    npad = EROWS * EB - E
    src = jnp.pad(edge_index[0], (0, npad)).reshape(EROWS, EB)
    pad_dst = N + jnp.arange(npad, dtype=edge_index.dtype) % (NP - N)
    dst = jnp.concatenate([edge_index[1], pad_dst]).reshape(EROWS, EB)
    batch3 = jnp.pad(batch, (0, NP - N),
                     constant_values=B).reshape(NBLK, 1, NP // NBLK)
    w_pad = jnp.pad(W_gcn, ((0, 0), (0, HP - W_gcn.shape[1])))
    bg = _row(jnp.pad(b_gcn, (0, HP - b_gcn.shape[0])), HP)
    h1 = bn2_gamma.shape[0]
    h2 = bn3_gamma.shape[0]
    g2 = _row(jnp.pad(bn2_gamma, (0, HP - h1)), HP)
    b2 = _row(jnp.pad(bn2_beta, (0, HP - h1)), HP)
    m2 = _row(jnp.pad(bn2_mean, (0, HP - h1)), HP)
    v2 = _row(jnp.pad(bn2_var, (0, HP - h1), constant_values=1.0), HP)
    w1 = jnp.pad(lin1_W, ((0, HP - h1), (0, HP - lin1_W.shape[1])))
    bb1 = _row(jnp.pad(lin1_b, (0, HP - lin1_b.shape[0])), HP)
    g3 = _row(jnp.pad(bn3_gamma, (0, HP - h2)), HP)
    b3 = _row(jnp.pad(bn3_beta, (0, HP - h2)), HP)
    m3 = _row(jnp.pad(bn3_mean, (0, HP - h2)), HP)
    v3 = _row(jnp.pad(bn3_var, (0, HP - h2), constant_values=1.0), HP)
    w2 = jnp.pad(lin2_W, ((0, HP - h2), (0, HP - lin2_W.shape[1])))
    bb2 = _row(jnp.pad(lin2_b, (0, HP - lin2_b.shape[0])), HP)
    bn1g, bn1b = bn1_gamma[None, :], bn1_beta[None, :]
    bn1m, bn1v = bn1_mean[None, :], bn1_var[None, :]
    zeros_nh = jnp.zeros((NP, HP), f32)

    deg_k, msg_k = _sc_kernels()
    deg_p = deg_k(dst)
    hw_s = _tc_prep(x_pad, bn1g, bn1b, bn1m, bn1v, w_pad, deg_p)
    p = msg_k(src, dst, hw_s, zeros_nh)
    out = _tc_head(p, deg_p, batch3, bg, g2, b2, m2, v2, w1, bb1,
                   g3, b3, m3, v3, w2, bb2)
    return out[:, :lin2_W.shape[1]]
